# baseline (speedup 1.0000x reference)
import jax
import jax.numpy as jnp
from jax.experimental import pallas as pl
from jax.experimental.pallas import tpu as pltpu
from jax.experimental.pallas import tpu_sc as plsc

N = 10000
K = 32
D = 128
NK = N * K
P = 200
R = P * K
GW = 128
NC = 10
CP = N // NC
CNK = CP * K
CPAD = 32768


def _ln(x, g, b, jmat, eps=1e-5):
    m = _dot(x, jmat)
    d = x - m
    v = _dot(d * d, jmat)
    s = jax.lax.rsqrt(v + eps)
    return (d * s) * g + b


def _leaky(x):
    return jnp.maximum(x, 0.2 * x)


def _dot(x, w):
    return jnp.dot(x, w, preferred_element_type=jnp.float32)


def _sc_gather(table, idx):
    mesh = plsc.VectorSubcoreMesh(core_axis_name="core", subcore_axis_name="subcore")

    @pl.kernel(
        out_type=jax.ShapeDtypeStruct((CPAD, table.shape[1]), table.dtype),
        mesh=mesh,
    )
    def kern(tab_hbm, i_hbm, o_hbm):
        def body(i_vmem, o_vmem):
            pltpu.sync_copy(tab_hbm.at[i_vmem.at[0]], o_vmem)

        pltpu.emit_pipeline(
            body,
            grid=(CPAD // GW,),
            in_specs=[pl.BlockSpec((1, GW), index_map=lambda i: (0, i))],
            out_specs=[pl.BlockSpec((GW, table.shape[1]), index_map=lambda i: (i, 0))],
            core_axis_name=("core", "subcore"),
            dimension_semantics=(pltpu.PARALLEL,),
        )(i_hbm, o_hbm)

    return kern(table, idx)


def _fused_body(geom_ref, s_ref,
                gW1_ref, gb1_ref, gg1_ref, gB1_ref,
                gW2_ref, gb2_ref, gg2_ref, gB2_ref,
                Wc_ref, bc_ref,
                fW1b_ref, fg1_ref, fB1_ref,
                fW2_ref, fb2_ref, fg2_ref, fB2_ref,
                fW3_ref, fb3_ref,
                j64_ref, j128_ref,
                out_ref):
    j64 = j64_ref[...]
    j128 = j128_ref[...]
    gp = geom_ref[...]
    gW1 = gW1_ref[...]
    h1 = jnp.concatenate(
        [_dot(gp[:, 4 * k:4 * k + 4], gW1) for k in range(K)], axis=0)
    h = _leaky(_ln(h1 + gb1_ref[...],
                   gg1_ref[...], gB1_ref[...], j64))
    h = _leaky(_ln(_dot(h, gW2_ref[...]) + gb2_ref[...],
                   gg2_ref[...], gB2_ref[...], j128))

    rel_term = _dot(s_ref[...], fW1b_ref[...])
    a1 = _dot(h, Wc_ref[...]) + rel_term + bc_ref[...]
    h = _leaky(_ln(a1, fg1_ref[...], fB1_ref[...], j64))
    h = _leaky(_ln(_dot(h, fW2_ref[...]) + fb2_ref[...],
                   fg2_ref[...], fB2_ref[...], j128))

    hm = jnp.mean(h.reshape(K, P, 128), axis=0)
    out_ref[...] = _dot(hm, fW3_ref[...]) + fb3_ref[...]


def _row2(v):
    return v.reshape(1, -1)


def kernel(point_features, geom_features, neighbor_idxs,
           gW1, gb1, gg1, gB1, gW2, gb2, gg2, gB2, gW3, gb3,
           fW1, fb1, fg1, fB1, fW2, fb2, fg2, fB2, fW3, fb3):
    pf = point_features.reshape(N, D)
    fW1a, fW1b = fW1[:D], fW1[D:]
    Wc = gW3 @ fW1a
    bc = gb3 @ fW1a + fb1

    idx = (neighbor_idxs.reshape(N // P, P, K)
           .transpose(0, 2, 1).reshape(-1).astype(jnp.int32))
    pad_rows = (jnp.arange(CPAD - CNK, dtype=jnp.int32) * 13) % N
    s_chunks = [
        _sc_gather(pf, jnp.concatenate(
            [jax.lax.dynamic_slice_in_dim(idx, c * CNK, CNK), pad_rows]
        ).reshape(1, CPAD))
        for c in range(NC)
    ]

    geom = geom_features.reshape(N, K * 4)

    wspec = lambda shape: pl.BlockSpec(shape, lambda i: (0, 0))
    outs = []
    for c in range(NC):
        off = c * (CP // P)
        in_specs = [
            pl.BlockSpec((P, K * 4), lambda i, off=off: (off + i, 0)),
            pl.BlockSpec((R, D), lambda i: (i, 0)),
            wspec((4, 64)), wspec((1, 64)), wspec((1, 64)), wspec((1, 64)),
            wspec((64, 128)), wspec((1, 128)), wspec((1, 128)), wspec((1, 128)),
            wspec((128, 64)), wspec((1, 64)),
            wspec((D, 64)), wspec((1, 64)), wspec((1, 64)),
            wspec((64, 128)), wspec((1, 128)), wspec((1, 128)), wspec((1, 128)),
            wspec((128, D)), wspec((1, D)),
            wspec((64, 64)), wspec((128, 128)),
        ]
        outs.append(pl.pallas_call(
            _fused_body,
            grid=(CP // P,),
            in_specs=in_specs,
            out_specs=pl.BlockSpec((P, D), lambda i: (i, 0)),
            out_shape=jax.ShapeDtypeStruct((CP, D), jnp.float32),
        )(geom, s_chunks[c],
          gW1, _row2(gb1), _row2(gg1), _row2(gB1),
          gW2, _row2(gb2), _row2(gg2), _row2(gB2),
          Wc, _row2(bc),
          fW1b, _row2(fg1), _row2(fB1),
          fW2, _row2(fb2), _row2(fg2), _row2(fB2),
          fW3, _row2(fb3),
          jnp.full((64, 64), 1.0 / 64, jnp.float32),
          jnp.full((128, 128), 1.0 / 128, jnp.float32)))
    return jnp.concatenate(outs, axis=0).reshape(1, N, D)


# speedup vs baseline: 1.0376x; 1.0376x over previous
import jax
import jax.numpy as jnp
from jax.experimental import pallas as pl
from jax.experimental.pallas import tpu as pltpu
from jax.experimental.pallas import tpu_sc as plsc

N = 10000
K = 32
D = 128
NK = N * K
P = 400
R = P * K
GW = 128
NC = 5
CP = N // NC
CNK = CP * K
CPAD = 65536


def _ln(x, g, b, jmat, eps=1e-5):
    m = _dot(x, jmat)
    d = x - m
    v = _dot(d * d, jmat)
    s = jax.lax.rsqrt(v + eps)
    return (d * s) * g + b


def _leaky(x):
    return jnp.maximum(x, 0.2 * x)


def _dot(x, w):
    return jnp.dot(x, w, preferred_element_type=jnp.float32)


def _sc_gather(table, idx):
    mesh = plsc.VectorSubcoreMesh(core_axis_name="core", subcore_axis_name="subcore")

    @pl.kernel(
        out_type=jax.ShapeDtypeStruct((CPAD, table.shape[1]), table.dtype),
        mesh=mesh,
    )
    def kern(tab_hbm, i_hbm, o_hbm):
        def body(i_vmem, o_vmem):
            pltpu.sync_copy(tab_hbm.at[i_vmem.at[0]], o_vmem)

        pltpu.emit_pipeline(
            body,
            grid=(CPAD // GW,),
            in_specs=[pl.BlockSpec((1, GW), index_map=lambda i: (0, i))],
            out_specs=[pl.BlockSpec((GW, table.shape[1]), index_map=lambda i: (i, 0))],
            core_axis_name=("core", "subcore"),
            dimension_semantics=(pltpu.PARALLEL,),
        )(i_hbm, o_hbm)

    return kern(table, idx)


def _fused_body(geom_ref, s_ref,
                gW1_ref, gb1_ref, gg1_ref, gB1_ref,
                gW2_ref, gb2_ref, gg2_ref, gB2_ref,
                Wc_ref, bc_ref,
                fW1b_ref, fg1_ref, fB1_ref,
                fW2_ref, fb2_ref, fg2_ref, fB2_ref,
                fW3_ref, fb3_ref,
                j64_ref, j128_ref,
                out_ref):
    j64 = j64_ref[...]
    j128 = j128_ref[...]
    gp = geom_ref[...]
    gW1 = gW1_ref[...]
    h1 = jnp.concatenate(
        [_dot(gp[:, 4 * k:4 * k + 4], gW1) for k in range(K)], axis=0)
    h = _leaky(_ln(h1 + gb1_ref[...],
                   gg1_ref[...], gB1_ref[...], j64))
    h = _leaky(_ln(_dot(h, gW2_ref[...]) + gb2_ref[...],
                   gg2_ref[...], gB2_ref[...], j128))

    rel_term = _dot(s_ref[...], fW1b_ref[...])
    a1 = _dot(h, Wc_ref[...]) + rel_term + bc_ref[...]
    h = _leaky(_ln(a1, fg1_ref[...], fB1_ref[...], j64))
    h = _leaky(_ln(_dot(h, fW2_ref[...]) + fb2_ref[...],
                   fg2_ref[...], fB2_ref[...], j128))

    hm = jnp.mean(h.reshape(K, P, 128), axis=0)
    out_ref[...] = _dot(hm, fW3_ref[...]) + fb3_ref[...]


def _row2(v):
    return v.reshape(1, -1)


def kernel(point_features, geom_features, neighbor_idxs,
           gW1, gb1, gg1, gB1, gW2, gb2, gg2, gB2, gW3, gb3,
           fW1, fb1, fg1, fB1, fW2, fb2, fg2, fB2, fW3, fb3):
    pf = point_features.reshape(N, D)
    fW1a, fW1b = fW1[:D], fW1[D:]
    Wc = gW3 @ fW1a
    bc = gb3 @ fW1a + fb1

    idx = (neighbor_idxs.reshape(N // P, P, K)
           .transpose(0, 2, 1).reshape(-1).astype(jnp.int32))
    pad_rows = (jnp.arange(CPAD - CNK, dtype=jnp.int32) * 13) % N
    s_chunks = [
        _sc_gather(pf, jnp.concatenate(
            [jax.lax.dynamic_slice_in_dim(idx, c * CNK, CNK), pad_rows]
        ).reshape(1, CPAD))
        for c in range(NC)
    ]

    geom = geom_features.reshape(N, K * 4)

    wspec = lambda shape: pl.BlockSpec(shape, lambda i: (0, 0))
    outs = []
    for c in range(NC):
        off = c * (CP // P)
        in_specs = [
            pl.BlockSpec((P, K * 4), lambda i, off=off: (off + i, 0)),
            pl.BlockSpec((R, D), lambda i: (i, 0)),
            wspec((4, 64)), wspec((1, 64)), wspec((1, 64)), wspec((1, 64)),
            wspec((64, 128)), wspec((1, 128)), wspec((1, 128)), wspec((1, 128)),
            wspec((128, 64)), wspec((1, 64)),
            wspec((D, 64)), wspec((1, 64)), wspec((1, 64)),
            wspec((64, 128)), wspec((1, 128)), wspec((1, 128)), wspec((1, 128)),
            wspec((128, D)), wspec((1, D)),
            wspec((64, 64)), wspec((128, 128)),
        ]
        outs.append(pl.pallas_call(
            _fused_body,
            grid=(CP // P,),
            in_specs=in_specs,
            out_specs=pl.BlockSpec((P, D), lambda i: (i, 0)),
            out_shape=jax.ShapeDtypeStruct((CP, D), jnp.float32),
        )(geom, s_chunks[c],
          gW1, _row2(gb1), _row2(gg1), _row2(gB1),
          gW2, _row2(gb2), _row2(gg2), _row2(gB2),
          Wc, _row2(bc),
          fW1b, _row2(fg1), _row2(fB1),
          fW2, _row2(fb2), _row2(fg2), _row2(fB2),
          fW3, _row2(fb3),
          jnp.full((64, 64), 1.0 / 64, jnp.float32),
          jnp.full((128, 128), 1.0 / 128, jnp.float32)))
    return jnp.concatenate(outs, axis=0).reshape(1, N, D)
